# baseline (speedup 1.0000x reference)
import functools

import jax
import jax.numpy as jnp
from jax import lax
from jax.experimental import pallas as pl
from jax.experimental.pallas import tpu as pltpu
from jax.experimental.pallas import tpu_sc as plsc

N_NODES = 10000
F = 128
FH = F // 2
N_EDGES = 320000

NC = 2
NS = 16
EPT = N_EDGES // NS
WIN = 128
NWIN = EPT // WIN
TAIL = EPT - NWIN * WIN
N_PAD = 10240
ZROWS = 128
ROWS_PT = N_PAD // NS
DEG_PAD = 10240
DEG_PT = DEG_PAD // NS

_mesh = plsc.VectorSubcoreMesh(core_axis_name="c", subcore_axis_name="s")

_sc_params = pltpu.CompilerParams(
    needs_layout_passes=False, use_tc_tiling_on_sc=False)


@functools.partial(
    pl.kernel,
    out_type=jax.ShapeDtypeStruct((NC, DEG_PAD), jnp.float32),
    mesh=_mesh,
    scratch_types=[
        pltpu.VMEM((EPT // 2,), jnp.int32),
        pltpu.VMEM((DEG_PAD,), jnp.float32),
        pltpu.VMEM((NS, DEG_PT), jnp.float32),
        pltpu.VMEM_SHARED((NS, DEG_PAD), jnp.float32),
    ],
    compiler_params=_sc_params,
)
def _deg_kernel(dst_hbm, deg_out, dst_v, hist_v, stripe_v, stage_sh):
    c = lax.axis_index("c")
    s = lax.axis_index("s")
    g = c * NS + s
    half = EPT // 2
    pltpu.sync_copy(dst_hbm.at[pl.ds(g * half, half)], dst_v)

    zero16 = jnp.zeros((16,), jnp.float32)
    ones16 = jnp.ones((16,), jnp.float32)

    @pl.loop(0, DEG_PAD, step=16)
    def _(i):
        hist_v[pl.ds(i, 16)] = zero16

    @pl.loop(0, half, step=16)
    def _(i):
        plsc.addupdate_scatter(hist_v, [dst_v[pl.ds(i, 16)]], ones16)

    pltpu.sync_copy(hist_v, stage_sh.at[s])
    plsc.subcore_barrier()
    for r in range(NS):
        pltpu.sync_copy(stage_sh.at[r, pl.ds(s * DEG_PT, DEG_PT)], stripe_v.at[r])

    @pl.loop(0, DEG_PT, step=16)
    def _(i):
        acc = stripe_v[0, pl.ds(i, 16)]
        for r in range(1, NS):
            acc = acc + stripe_v[r, pl.ds(i, 16)]
        stripe_v[0, pl.ds(i, 16)] = acc

    pltpu.sync_copy(stripe_v.at[0], deg_out.at[c, pl.ds(s * DEG_PT, DEG_PT)])


_NBUF = 4
_LOOK = 3


@functools.partial(
    pl.kernel,
    out_type=jax.ShapeDtypeStruct((N_PAD, NC, FH), jnp.float32),
    mesh=_mesh,
    scratch_types=[
        pltpu.VMEM((EPT,), jnp.int32),
        pltpu.VMEM((EPT,), jnp.int32),
        pltpu.VMEM((WIN,), jnp.int32),
        pltpu.VMEM((TAIL,), jnp.int32),
        pltpu.VMEM((_NBUF, WIN, FH), jnp.float32),
        pltpu.VMEM((ZROWS, FH), jnp.float32),
        pltpu.VMEM_SHARED((N_PAD, FH), jnp.float32),
    ]
    + [pltpu.SemaphoreType.DMA] * _NBUF,
    compiler_params=_sc_params,
)
def _agg_kernel(hp_hbm, src_hbm, dst_hbm, out_hbm,
                src_v, dst_v, dwin_v, dtail_v, rows_v, zbuf, agg_sh, *sems):
    gsem = sems
    c = lax.axis_index("c")
    s = lax.axis_index("s")
    pltpu.sync_copy(src_hbm.at[pl.ds(s * EPT, EPT)], src_v)
    pltpu.sync_copy(dst_hbm.at[pl.ds(s * EPT, EPT)], dst_v)

    @pl.loop(0, EPT, step=16)
    def _(i):
        src_v[pl.ds(i, 16)] = src_v[pl.ds(i, 16)] * 2 + c

    zero16 = jnp.zeros((16,), jnp.float32)

    @pl.loop(0, ZROWS)
    def _(i):
        for cc in range(FH // 16):
            zbuf[i, pl.ds(cc * 16, 16)] = zero16

    for kk in range(ROWS_PT // ZROWS):
        pltpu.sync_copy(zbuf, agg_sh.at[pl.ds(s * ROWS_PT + kk * ZROWS, ZROWS)])
    plsc.subcore_barrier()

    def g_start(w, j):
        pltpu.async_copy(hp_hbm.at[src_v.at[pl.ds(w * WIN, WIN)]],
                         rows_v.at[j], gsem[j])

    def g_wait(w, j):
        pltpu.make_async_copy(hp_hbm.at[src_v.at[pl.ds(w * WIN, WIN)]],
                              rows_v.at[j], gsem[j]).wait()

    def scat(w, j):
        for i in range(WIN // 16):
            dwin_v[pl.ds(16 * i, 16)] = dst_v[pl.ds(w * WIN + 16 * i, 16)]
        pltpu.sync_copy(rows_v.at[j], agg_sh.at[dwin_v], add=True)

    for w in range(_LOOK):
        g_start(w, w)
    g_start(_LOOK, _LOOK)
    g_wait(0, 0)
    scat(0, 0)

    @pl.loop(0, (NWIN - _NBUF) // _NBUF)
    def _(k):
        w0 = _NBUF * k + 1
        for t in range(_NBUF):
            w = w0 + t
            j = (1 + t) % _NBUF
            g_start(w + _LOOK, (1 + t + _LOOK) % _NBUF)
            g_wait(w, j)
            scat(w, j)

    for w in range(NWIN - _LOOK, NWIN):
        j = w % _NBUF
        g_wait(w, j)
        scat(w, j)

    tb = NWIN * WIN
    for i in range(TAIL // 16):
        dtail_v[pl.ds(16 * i, 16)] = dst_v[pl.ds(tb + 16 * i, 16)]
    pltpu.make_async_copy(hp_hbm.at[src_v.at[pl.ds(tb, TAIL)]],
                          rows_v.at[0].at[pl.ds(0, TAIL)], gsem[0]).start()
    pltpu.make_async_copy(hp_hbm.at[src_v.at[pl.ds(tb, TAIL)]],
                          rows_v.at[0].at[pl.ds(0, TAIL)], gsem[0]).wait()
    pltpu.sync_copy(rows_v.at[0].at[pl.ds(0, TAIL)],
                    agg_sh.at[dtail_v], add=True)
    plsc.subcore_barrier()

    for kk in range(ROWS_PT // ZROWS):
        off = s * ROWS_PT + kk * ZROWS
        pltpu.sync_copy(agg_sh.at[pl.ds(off, ZROWS)],
                        out_hbm.at[pl.ds(off, ZROWS), c])


_R = 1000


def _mm_scale(x, w_t, b, d0, d1):
    def body(x_ref, w_ref, b_ref, d0_ref, d1_ref, o_ref):
        dinv = lax.rsqrt(d0_ref[...] + d1_ref[...] + 1.0)
        o_ref[...] = dinv * (
            jnp.dot(x_ref[...], w_ref[...], preferred_element_type=jnp.float32)
            + b_ref[...])

    return pl.pallas_call(
        body,
        grid=(N_NODES // _R,),
        in_specs=[
            pl.BlockSpec((_R, F), lambda i: (i, 0)),
            pl.BlockSpec((F, F), lambda i: (0, 0)),
            pl.BlockSpec((1, F), lambda i: (0, 0)),
            pl.BlockSpec((_R, 1), lambda i: (i, 0)),
            pl.BlockSpec((_R, 1), lambda i: (i, 0)),
        ],
        out_specs=pl.BlockSpec((_R, F), lambda i: (i, 0)),
        out_shape=jax.ShapeDtypeStruct((N_NODES, F), jnp.float32),
    )(x, w_t, b, d0, d1)


def _relu_comb_mm(a, hp, w_t, b, d0, d1):
    def body(a_ref, hp_ref, w_ref, b_ref, d0_ref, d1_ref, o_ref):
        dinv = lax.rsqrt(d0_ref[...] + d1_ref[...] + 1.0)
        sblk = jnp.maximum(dinv * (a_ref[...] + hp_ref[...]), 0.0)
        o_ref[...] = dinv * (
            jnp.dot(sblk, w_ref[...], preferred_element_type=jnp.float32)
            + b_ref[...])

    return pl.pallas_call(
        body,
        grid=(N_NODES // _R,),
        in_specs=[
            pl.BlockSpec((_R, F), lambda i: (i, 0)),
            pl.BlockSpec((_R, F), lambda i: (i, 0)),
            pl.BlockSpec((F, F), lambda i: (0, 0)),
            pl.BlockSpec((1, F), lambda i: (0, 0)),
            pl.BlockSpec((_R, 1), lambda i: (i, 0)),
            pl.BlockSpec((_R, 1), lambda i: (i, 0)),
        ],
        out_specs=pl.BlockSpec((_R, F), lambda i: (i, 0)),
        out_shape=jax.ShapeDtypeStruct((N_NODES, F), jnp.float32),
    )(a, hp, w_t, b, d0, d1)


def _final_comb(a, hp, d0, d1):
    def body(a_ref, hp_ref, d0_ref, d1_ref, o_ref):
        dinv = lax.rsqrt(d0_ref[...] + d1_ref[...] + 1.0)
        o_ref[...] = dinv * (a_ref[...] + hp_ref[...])

    return pl.pallas_call(
        body,
        grid=(N_NODES // _R,),
        in_specs=[
            pl.BlockSpec((_R, F), lambda i: (i, 0)),
            pl.BlockSpec((_R, F), lambda i: (i, 0)),
            pl.BlockSpec((_R, 1), lambda i: (i, 0)),
            pl.BlockSpec((_R, 1), lambda i: (i, 0)),
        ],
        out_specs=pl.BlockSpec((_R, F), lambda i: (i, 0)),
        out_shape=jax.ShapeDtypeStruct((N_NODES, F), jnp.float32),
    )(a, hp, d0, d1)


def kernel(x, ei, W1, b1, W2, b2):
    ei = ei.astype(jnp.int32)
    src, dst = ei[0], ei[1]

    deg = _deg_kernel(dst)
    d0 = deg[0, :N_NODES].reshape(N_NODES, 1)
    d1 = deg[1, :N_NODES].reshape(N_NODES, 1)

    hp1 = _mm_scale(x, W1.T, b1.reshape(1, F), d0, d1)
    a1 = _agg_kernel(hp1.reshape(2 * N_NODES, FH), src, dst)
    hp2 = _relu_comb_mm(a1.reshape(N_PAD, F), hp1,
                        W2.T, b2.reshape(1, F), d0, d1)
    a2 = _agg_kernel(hp2.reshape(2 * N_NODES, FH), src, dst)
    return _final_comb(a2.reshape(N_PAD, F), hp2, d0, d1)


# speedup vs baseline: 1.3361x; 1.3361x over previous
import functools

import jax
import jax.numpy as jnp
from jax import lax
from jax.experimental import pallas as pl
from jax.experimental.pallas import tpu as pltpu
from jax.experimental.pallas import tpu_sc as plsc

N_NODES = 10000
F = 128
FH = F // 2
N_EDGES = 320000

NC = 2
NS = 16
EPT = N_EDGES // NS
WIN = 128
NWIN = EPT // WIN
TAIL = EPT - NWIN * WIN
N_PAD = 10240
ZROWS = 128
ROWS_PT = N_PAD // NS
DEG_PAD = 10240
DEG_PT = DEG_PAD // NS

_mesh = plsc.VectorSubcoreMesh(core_axis_name="c", subcore_axis_name="s")

_sc_params = pltpu.CompilerParams(
    needs_layout_passes=False, use_tc_tiling_on_sc=False)


@functools.partial(
    pl.kernel,
    out_type=jax.ShapeDtypeStruct((NC, DEG_PAD), jnp.float32),
    mesh=_mesh,
    scratch_types=[
        pltpu.VMEM((EPT // 2,), jnp.int32),
        pltpu.VMEM((DEG_PAD,), jnp.float32),
        pltpu.VMEM((NS, DEG_PT), jnp.float32),
        pltpu.VMEM_SHARED((NS, DEG_PAD), jnp.float32),
    ],
    compiler_params=_sc_params,
)
def _deg_kernel(ei_hbm, deg_out, dst_v, hist_v, stripe_v, stage_sh):
    c = lax.axis_index("c")
    s = lax.axis_index("s")
    g = c * NS + s
    half = EPT // 2
    pltpu.sync_copy(ei_hbm.at[1, pl.ds(g * half, half)], dst_v)

    zero16 = jnp.zeros((16,), jnp.float32)
    ones16 = jnp.ones((16,), jnp.float32)

    @pl.loop(0, DEG_PAD, step=16)
    def _(i):
        hist_v[pl.ds(i, 16)] = zero16

    @pl.loop(0, half, step=16)
    def _(i):
        plsc.addupdate_scatter(hist_v, [dst_v[pl.ds(i, 16)]], ones16)

    pltpu.sync_copy(hist_v, stage_sh.at[s])
    plsc.subcore_barrier()
    for r in range(NS):
        pltpu.sync_copy(stage_sh.at[r, pl.ds(s * DEG_PT, DEG_PT)], stripe_v.at[r])

    @pl.loop(0, DEG_PT, step=16)
    def _(i):
        acc = stripe_v[0, pl.ds(i, 16)]
        for r in range(1, NS):
            acc = acc + stripe_v[r, pl.ds(i, 16)]
        stripe_v[0, pl.ds(i, 16)] = acc

    pltpu.sync_copy(stripe_v.at[0], deg_out.at[c, pl.ds(s * DEG_PT, DEG_PT)])


_NBUF = 4
_LOOK = 3


@functools.partial(
    pl.kernel,
    out_type=jax.ShapeDtypeStruct((N_PAD, F), jnp.float32),
    mesh=_mesh,
    scratch_types=[
        pltpu.VMEM((EPT,), jnp.int32),
        pltpu.VMEM((EPT,), jnp.int32),
        pltpu.VMEM((WIN,), jnp.int32),
        pltpu.VMEM((TAIL,), jnp.int32),
        pltpu.VMEM((_NBUF, WIN, FH), jnp.float32),
        pltpu.VMEM((ZROWS, FH), jnp.float32),
        pltpu.VMEM_SHARED((N_PAD, FH), jnp.float32),
    ]
    + [pltpu.SemaphoreType.DMA] * _NBUF,
    compiler_params=_sc_params,
)
def _agg_kernel(hp_hbm, ei_hbm, out_hbm,
                src_v, dst_v, dwin_v, dtail_v, rows_v, zbuf, agg_sh, *sems):
    gsem = sems
    c = lax.axis_index("c")
    s = lax.axis_index("s")
    pltpu.sync_copy(ei_hbm.at[0, pl.ds(s * EPT, EPT)], src_v)
    pltpu.sync_copy(ei_hbm.at[1, pl.ds(s * EPT, EPT)], dst_v)

    @pl.loop(0, EPT, step=16)
    def _(i):
        src_v[pl.ds(i, 16)] = src_v[pl.ds(i, 16)] * 2 + c

    zero16 = jnp.zeros((16,), jnp.float32)

    @pl.loop(0, ZROWS)
    def _(i):
        for cc in range(FH // 16):
            zbuf[i, pl.ds(cc * 16, 16)] = zero16

    for kk in range(ROWS_PT // ZROWS):
        pltpu.sync_copy(zbuf, agg_sh.at[pl.ds(s * ROWS_PT + kk * ZROWS, ZROWS)])
    plsc.subcore_barrier()

    def g_start(w, j):
        pltpu.async_copy(hp_hbm.at[src_v.at[pl.ds(w * WIN, WIN)]],
                         rows_v.at[j], gsem[j])

    def g_wait(w, j):
        pltpu.make_async_copy(hp_hbm.at[src_v.at[pl.ds(w * WIN, WIN)]],
                              rows_v.at[j], gsem[j]).wait()

    def scat(w, j):
        for i in range(WIN // 16):
            dwin_v[pl.ds(16 * i, 16)] = dst_v[pl.ds(w * WIN + 16 * i, 16)]
        pltpu.sync_copy(rows_v.at[j], agg_sh.at[dwin_v], add=True)

    for w in range(_LOOK):
        g_start(w, w)
    g_start(_LOOK, _LOOK)
    g_wait(0, 0)
    scat(0, 0)

    @pl.loop(0, (NWIN - _NBUF) // _NBUF)
    def _(k):
        w0 = _NBUF * k + 1
        for t in range(_NBUF):
            w = w0 + t
            j = (1 + t) % _NBUF
            g_start(w + _LOOK, (1 + t + _LOOK) % _NBUF)
            g_wait(w, j)
            scat(w, j)

    for w in range(NWIN - _LOOK, NWIN):
        j = w % _NBUF
        g_wait(w, j)
        scat(w, j)

    tb = NWIN * WIN
    for i in range(TAIL // 16):
        dtail_v[pl.ds(16 * i, 16)] = dst_v[pl.ds(tb + 16 * i, 16)]
    pltpu.make_async_copy(hp_hbm.at[src_v.at[pl.ds(tb, TAIL)]],
                          rows_v.at[0].at[pl.ds(0, TAIL)], gsem[0]).start()
    pltpu.make_async_copy(hp_hbm.at[src_v.at[pl.ds(tb, TAIL)]],
                          rows_v.at[0].at[pl.ds(0, TAIL)], gsem[0]).wait()
    pltpu.sync_copy(rows_v.at[0].at[pl.ds(0, TAIL)],
                    agg_sh.at[dtail_v], add=True)
    plsc.subcore_barrier()

    for kk in range(ROWS_PT // ZROWS):
        off = s * ROWS_PT + kk * ZROWS
        pltpu.sync_copy(agg_sh.at[pl.ds(off, ZROWS)],
                        out_hbm.at[pl.ds(off, ZROWS), pl.ds(c * FH, FH)])


_R = 1000


def _mm_scale(x, w_t, b, d0, d1):
    def body(x_ref, w_ref, b_ref, d0_ref, d1_ref, o_ref):
        dinv = lax.rsqrt(d0_ref[...] + d1_ref[...] + 1.0)
        o_ref[...] = dinv * (
            jnp.dot(x_ref[...], w_ref[...], preferred_element_type=jnp.float32)
            + b_ref[...])

    return pl.pallas_call(
        body,
        grid=(N_NODES // _R,),
        in_specs=[
            pl.BlockSpec((_R, F), lambda i: (i, 0)),
            pl.BlockSpec((F, F), lambda i: (0, 0)),
            pl.BlockSpec((1, F), lambda i: (0, 0)),
            pl.BlockSpec((_R, 1), lambda i: (i, 0)),
            pl.BlockSpec((_R, 1), lambda i: (i, 0)),
        ],
        out_specs=pl.BlockSpec((_R, F), lambda i: (i, 0)),
        out_shape=jax.ShapeDtypeStruct((N_NODES, F), jnp.float32),
    )(x, w_t, b, d0, d1)


def _relu_comb_mm(a, hp, w_t, b, d0, d1):
    def body(a_ref, hp_ref, w_ref, b_ref, d0_ref, d1_ref, o_ref):
        dinv = lax.rsqrt(d0_ref[...] + d1_ref[...] + 1.0)
        sblk = jnp.maximum(dinv * (a_ref[...] + hp_ref[...]), 0.0)
        o_ref[...] = dinv * (
            jnp.dot(sblk, w_ref[...], preferred_element_type=jnp.float32)
            + b_ref[...])

    return pl.pallas_call(
        body,
        grid=(N_NODES // _R,),
        in_specs=[
            pl.BlockSpec((_R, F), lambda i: (i, 0)),
            pl.BlockSpec((_R, F), lambda i: (i, 0)),
            pl.BlockSpec((F, F), lambda i: (0, 0)),
            pl.BlockSpec((1, F), lambda i: (0, 0)),
            pl.BlockSpec((_R, 1), lambda i: (i, 0)),
            pl.BlockSpec((_R, 1), lambda i: (i, 0)),
        ],
        out_specs=pl.BlockSpec((_R, F), lambda i: (i, 0)),
        out_shape=jax.ShapeDtypeStruct((N_NODES, F), jnp.float32),
    )(a, hp, w_t, b, d0, d1)


def _final_comb(a, hp, d0, d1):
    def body(a_ref, hp_ref, d0_ref, d1_ref, o_ref):
        dinv = lax.rsqrt(d0_ref[...] + d1_ref[...] + 1.0)
        o_ref[...] = dinv * (a_ref[...] + hp_ref[...])

    return pl.pallas_call(
        body,
        grid=(N_NODES // _R,),
        in_specs=[
            pl.BlockSpec((_R, F), lambda i: (i, 0)),
            pl.BlockSpec((_R, F), lambda i: (i, 0)),
            pl.BlockSpec((_R, 1), lambda i: (i, 0)),
            pl.BlockSpec((_R, 1), lambda i: (i, 0)),
        ],
        out_specs=pl.BlockSpec((_R, F), lambda i: (i, 0)),
        out_shape=jax.ShapeDtypeStruct((N_NODES, F), jnp.float32),
    )(a, hp, d0, d1)


def kernel(x, ei, W1, b1, W2, b2):
    ei = ei.astype(jnp.int32)

    deg = _deg_kernel(ei)
    d0 = deg[0, :N_NODES].reshape(N_NODES, 1)
    d1 = deg[1, :N_NODES].reshape(N_NODES, 1)

    hp1 = _mm_scale(x, W1.T, b1.reshape(1, F), d0, d1)
    a1 = _agg_kernel(hp1.reshape(2 * N_NODES, FH), ei)
    hp2 = _relu_comb_mm(a1, hp1, W2.T, b2.reshape(1, F), d0, d1)
    a2 = _agg_kernel(hp2.reshape(2 * N_NODES, FH), ei)
    return _final_comb(a2, hp2, d0, d1)
